# baseline (speedup 1.0000x reference)
import functools

import jax
import jax.numpy as jnp
from jax import lax
from jax.experimental import pallas as pl
from jax.experimental.pallas import tpu as pltpu
from jax.experimental.pallas import tpu_sc as plsc

VOCAB = 1000000
D = 64
B = 16384
NUM_NS = 5

NC = 2
NS = 16
NW = NC * NS
B_PER_W = B // NW
CB = 128
NCHUNK = B_PER_W // CB
L = 16
BG = CB // L
OUT_PER_GROUP = L * NUM_NS


def _w2v_body(inp_hbm, tar_hbm, u_hbm, v_hbm, out_hbm,
              idx_u, idx_v, u_rows, v_rows, buf, out_c, sem):
    wid = lax.axis_index("s") * NC + lax.axis_index("c")
    base_b = wid * B_PER_W

    pltpu.sync_copy(inp_hbm.at[pl.ds(base_b, B_PER_W)], idx_u)
    pltpu.sync_copy(tar_hbm.at[pl.ds(base_b * NUM_NS, B_PER_W * NUM_NS)], idx_v)

    iota = lax.iota(jnp.int32, L)

    @pl.loop(0, NCHUNK)
    def _chunk(c):
        cb = c * CB
        cps = [pltpu.async_copy(u_hbm.at[idx_u.at[pl.ds(cb, CB)]], u_rows, sem)]
        for g in range(NUM_NS):
            cps.append(pltpu.async_copy(
                v_hbm.at[idx_v.at[pl.ds(cb * NUM_NS + g * CB, CB)]],
                v_rows.at[pl.ds(g * CB, CB)], sem))
        for cp in cps:
            cp.wait()

        @pl.loop(0, BG)
        def _group(bg):
            def row_parts(ref, r):
                parts = []
                for h in range(2):
                    wi = plsc.bitcast(ref[r, pl.ds(h * L, L)], jnp.int32)
                    parts.append(plsc.bitcast(wi << 16, jnp.float32))
                    parts.append(plsc.bitcast(
                        wi & jnp.int32(-65536), jnp.float32))
                return parts

            for bl in range(L):
                b = bg * L + bl
                u_parts = row_parts(u_rows, b)
                for j in range(NUM_NS):
                    r = b * NUM_NS + j
                    v_parts = row_parts(v_rows, r)
                    t = u_parts[0] * v_parts[0]
                    for p in range(1, 4):
                        t = t + u_parts[p] * v_parts[p]
                    buf[pl.ds((bl * NUM_NS + j) * L, L)] = t
            for og in range(NUM_NS):
                base_idx = iota * L + og * (L * L)
                acc = plsc.load_gather(buf, [base_idx])
                for l in range(1, L):
                    acc = acc + plsc.load_gather(buf, [base_idx + l])
                out_c[pl.ds(bg * OUT_PER_GROUP + og * L, L)] = acc

        pltpu.sync_copy(
            out_c,
            out_hbm.at[pl.ds((base_b + c * CB) * NUM_NS, CB * NUM_NS)])


@jax.jit
def _w2v(inp_flat, tar_flat, U, V):
    mesh = plsc.VectorSubcoreMesh(core_axis_name="c", subcore_axis_name="s")
    k = functools.partial(
        pl.kernel,
        out_type=jax.ShapeDtypeStruct((B * NUM_NS,), jnp.float32),
        mesh=mesh,
        scratch_types=[
            pltpu.VMEM((B_PER_W,), jnp.int32),
            pltpu.VMEM((B_PER_W * NUM_NS,), jnp.int32),
            pltpu.VMEM((CB, DW), jnp.float32),
            pltpu.VMEM((CB * NUM_NS, DW), jnp.float32),
            pltpu.VMEM((OUT_PER_GROUP * L,), jnp.float32),
            pltpu.VMEM((CB * NUM_NS,), jnp.float32),
            pltpu.SemaphoreType.DMA,
        ],
        compiler_params=pltpu.CompilerParams(
            needs_layout_passes=False, use_tc_tiling_on_sc=False),
    )(_w2v_body)
    return k(inp_flat, tar_flat, U, V)


TP_BC = 16384
TP_Q = TP_BC // 4
TP_GRID = -(-VOCAB // TP_BC)
TP_ROWS = TP_GRID * TP_Q
VPAD = TP_ROWS * 4
DW = D // 2


def _tp_body(x_ref, o_ref):
    eye = jnp.eye(D, dtype=jnp.float32)

    for s in range(4):
        y = jax.lax.dot_general(
            x_ref[:, s * TP_Q:(s + 1) * TP_Q], eye, (((0,), (0,)), ((), ())),
            preferred_element_type=jnp.float32)
        u = jax.lax.bitcast_convert_type(y, jnp.uint32) + jnp.uint32(0x8000)
        lo = u[:, 0:DW] >> 16
        hi = u[:, DW:D] & jnp.uint32(0xFFFF0000)
        o_ref[:, s * DW:(s + 1) * DW] = jax.lax.bitcast_convert_type(
            lo | hi, jnp.float32)


def _tc_transpose(xt):
    w = pl.pallas_call(
        _tp_body,
        grid=(TP_GRID,),
        in_specs=[pl.BlockSpec((D, TP_BC), lambda i: (0, i))],
        out_specs=pl.BlockSpec((TP_Q, 4 * DW), lambda i: (i, 0)),
        out_shape=jax.ShapeDtypeStruct((TP_ROWS, 4 * DW), jnp.float32),
    )(xt)
    return w.reshape(VPAD, DW)


def _permute_idx(v):
    return (v & ~(TP_BC - 1)) | ((v & (TP_Q - 1)) << 2) | ((v >> 12) & 3)


def kernel(inp, tar, U, V):
    inp_flat = _permute_idx(inp.reshape(-1).astype(jnp.int32))
    tar_flat = _permute_idx(tar.reshape(-1).astype(jnp.int32))
    u_rm = _tc_transpose(jnp.swapaxes(U, 0, 1))
    v_rm = _tc_transpose(jnp.swapaxes(V, 0, 1))
    out_flat = _w2v(inp_flat, tar_flat, u_rm, v_rm)
    return out_flat.reshape(B, NUM_NS)


# speedup vs baseline: 1.3939x; 1.3939x over previous
import functools

import jax
import jax.numpy as jnp
from jax import lax
from jax.experimental import pallas as pl
from jax.experimental.pallas import tpu as pltpu
from jax.experimental.pallas import tpu_sc as plsc

VOCAB = 1000000
D = 64
B = 16384
NUM_NS = 5

NC = 2
NS = 16
NW = NC * NS
B_PER_W = B // NW
CB = 128
NCHUNK = B_PER_W // CB
L = 16
BG = CB // L
OUT_PER_GROUP = L * NUM_NS


def _w2v_body(inp_hbm, tar_hbm, u_hbm, v_hbm, out_hbm,
              idx_u, idx_v, u_rows, v_rows, buf, out_c, sem):
    wid = lax.axis_index("s") * NC + lax.axis_index("c")
    base_b = wid * B_PER_W

    pltpu.sync_copy(inp_hbm.at[pl.ds(base_b, B_PER_W)], idx_u)
    pltpu.sync_copy(tar_hbm.at[pl.ds(base_b * NUM_NS, B_PER_W * NUM_NS)], idx_v)

    iota = lax.iota(jnp.int32, L)

    @pl.loop(0, NCHUNK)
    def _chunk(c):
        cb = c * CB
        cps = [pltpu.async_copy(u_hbm.at[idx_u.at[pl.ds(cb, CB)]], u_rows, sem)]
        for g in range(NUM_NS):
            cps.append(pltpu.async_copy(
                v_hbm.at[idx_v.at[pl.ds(cb * NUM_NS + g * CB, CB)]],
                v_rows.at[pl.ds(g * CB, CB)], sem))
        for cp in cps:
            cp.wait()

        @pl.loop(0, BG)
        def _group(bg):
            for bl in range(L):
                b = bg * L + bl
                u_parts = [u_rows[b, pl.ds(p * L, L)] for p in range(4)]
                for j in range(NUM_NS):
                    r = b * NUM_NS + j
                    t = u_parts[0] * v_rows[r, pl.ds(0, L)]
                    for p in range(1, 4):
                        t = t + u_parts[p] * v_rows[r, pl.ds(p * L, L)]
                    buf[pl.ds((bl * NUM_NS + j) * L, L)] = t
            for og in range(NUM_NS):
                base_idx = iota * L + og * (L * L)
                acc = plsc.load_gather(buf, [base_idx])
                for l in range(1, L):
                    acc = acc + plsc.load_gather(buf, [base_idx + l])
                out_c[pl.ds(bg * OUT_PER_GROUP + og * L, L)] = acc

        pltpu.sync_copy(
            out_c,
            out_hbm.at[pl.ds((base_b + c * CB) * NUM_NS, CB * NUM_NS)])


@jax.jit
def _w2v(inp_flat, tar_flat, U, V):
    mesh = plsc.VectorSubcoreMesh(core_axis_name="c", subcore_axis_name="s")
    k = functools.partial(
        pl.kernel,
        out_type=jax.ShapeDtypeStruct((B * NUM_NS,), jnp.float32),
        mesh=mesh,
        scratch_types=[
            pltpu.VMEM((B_PER_W,), jnp.int32),
            pltpu.VMEM((B_PER_W * NUM_NS,), jnp.int32),
            pltpu.VMEM((CB, D), jnp.float32),
            pltpu.VMEM((CB * NUM_NS, D), jnp.float32),
            pltpu.VMEM((OUT_PER_GROUP * L,), jnp.float32),
            pltpu.VMEM((CB * NUM_NS,), jnp.float32),
            pltpu.SemaphoreType.DMA,
        ],
        compiler_params=pltpu.CompilerParams(
            needs_layout_passes=False, use_tc_tiling_on_sc=False),
    )(_w2v_body)
    return k(inp_flat, tar_flat, U, V)


TP_BC = 32768
TP_H = TP_BC // 2
TP_GRID = -(-VOCAB // TP_BC)
TP_ROWS = TP_GRID * TP_H
VPAD = TP_ROWS * 2


def _tp_body(x_ref, o_ref):
    eye = jnp.eye(D, dtype=jnp.float32)
    o_ref[:, 0:D] = jax.lax.dot_general(
        x_ref[:, 0:TP_H], eye, (((0,), (0,)), ((), ())),
        preferred_element_type=jnp.float32)
    o_ref[:, D:2 * D] = jax.lax.dot_general(
        x_ref[:, TP_H:TP_BC], eye, (((0,), (0,)), ((), ())),
        preferred_element_type=jnp.float32)


def _tc_transpose(xt):
    w = pl.pallas_call(
        _tp_body,
        grid=(TP_GRID,),
        in_specs=[pl.BlockSpec((D, TP_BC), lambda i: (0, i))],
        out_specs=pl.BlockSpec((TP_H, 2 * D), lambda i: (i, 0)),
        out_shape=jax.ShapeDtypeStruct((TP_ROWS, 2 * D), jnp.float32),
    )(xt)
    return w.reshape(VPAD, D)


_TP_HS = TP_H.bit_length() - 1


def _permute_idx(v):
    return (v & ~(TP_BC - 1)) | ((v & (TP_H - 1)) << 1) | ((v >> _TP_HS) & 1)


def kernel(inp, tar, U, V):
    inp_flat = _permute_idx(inp.reshape(-1).astype(jnp.int32))
    tar_flat = _permute_idx(tar.reshape(-1).astype(jnp.int32))
    u_rm = _tc_transpose(jnp.swapaxes(U, 0, 1))
    v_rm = _tc_transpose(jnp.swapaxes(V, 0, 1))
    out_flat = _w2v(inp_flat, tar_flat, u_rm, v_rm)
    return out_flat.reshape(B, NUM_NS)
